# Initial kernel scaffold; baseline (speedup 1.0000x reference)
import jax
import jax.numpy as jnp
from jax.experimental import pallas as pl


def kernel(x, tw, tb, ew, eb, dw, db, ow, ob, cb_t, cb_e, cb_d0, cb_d1):
    raise NotImplementedError("write your pallas kernel here")


# speedup vs baseline: 1.4391x; 1.4391x over previous
import jax
import jax.numpy as jnp
from jax.experimental import pallas as pl
from jax.experimental.pallas import tpu as pltpu

B, T, D = 8, 2048, 512
N = B * T
KT, KE, KD = 1024, 512, 1024
TB = 1024
GRID = N // TB
COMMIT = 0.25

_F32 = jnp.float32


def _argmin_rows(d):
    dmin = jnp.min(d, axis=1, keepdims=True)
    k = d.shape[1]
    iota = jax.lax.broadcasted_iota(jnp.int32, d.shape, 1)
    big = jnp.int32(k)
    cand = jnp.where(d == dmin, iota, big)
    return jnp.min(cand, axis=1)


def _vq_block(f, cb):
    rowsq = jnp.sum(f * f, axis=1, keepdims=True)
    cbsq = jnp.sum(cb * cb, axis=1)
    s = jax.lax.dot_general(f, cb, (((1,), (1,)), ((), ())),
                            preferred_element_type=_F32)
    d = rowsq + cbsq[None, :] - 2.0 * s
    idx = _argmin_rows(d)
    oh = (jax.lax.broadcasted_iota(jnp.int32, d.shape, 1)
          == idx[:, None]).astype(_F32)
    q = jax.lax.dot_general(oh, cb, (((1,), (0,)), ((), ())),
                            preferred_element_type=_F32,
                            precision=jax.lax.Precision.HIGHEST)
    return idx, q


def _body(x_ref, tw_ref, tb_ref, ew_ref, eb_ref, dw_ref, db_ref,
          ow_ref, ob_ref, cbt_ref, cbe_ref, cbd0_ref, cbd1_ref,
          out_ref, it_ref, ie_ref, i0_ref, i1_ref, loss_ref):
    xb = x_ref[...]

    tf = jnp.dot(xb, tw_ref[...], preferred_element_type=_F32) + tb_ref[...]
    ef = jnp.dot(xb, ew_ref[...], preferred_element_type=_F32) + eb_ref[...]
    df = jnp.dot(xb, dw_ref[...], preferred_element_type=_F32) + db_ref[...]

    idx_t, q_t = _vq_block(tf, cbt_ref[...])
    idx_e, q_e = _vq_block(ef, cbe_ref[...])

    idx0, q0 = _vq_block(df, cbd0_ref[...])
    q0st = df + (q0 - df)
    r = df - q0st
    idx1, q1 = _vq_block(r, cbd1_ref[...])
    q1st = r + (q1 - r)

    tqst = tf + (q_t - tf)
    eqst = ef + (q_e - ef)
    dq = q0st + q1st

    ow = ow_ref[...]
    out = (jnp.dot(tqst, ow[0:D], preferred_element_type=_F32)
           + jnp.dot(eqst, ow[D:2 * D], preferred_element_type=_F32)
           + jnp.dot(dq, ow[2 * D:3 * D], preferred_element_type=_F32)
           + ob_ref[...])
    out_ref[...] = out

    it_ref[...] = idx_t.reshape(TB // 128, 128)
    ie_ref[...] = idx_e.reshape(TB // 128, 128)
    i0_ref[...] = idx0.reshape(TB // 128, 128)
    i1_ref[...] = idx1.reshape(TB // 128, 128)

    part = (jnp.sum((q_t - tf) ** 2) + jnp.sum((q_e - ef) ** 2)
            + jnp.sum((q0 - df) ** 2) + jnp.sum((q1 - r) ** 2))

    part2d = part.reshape(1, 1)

    @pl.when(pl.program_id(0) == 0)
    def _init():
        loss_ref[...] = part2d

    @pl.when(pl.program_id(0) != 0)
    def _acc():
        loss_ref[...] += part2d


def kernel(x, tw, tb, ew, eb, dw, db, ow, ob, cb_t, cb_e, cb_d0, cb_d1):
    xf = x.reshape(N, D)
    whole = lambda shape: pl.BlockSpec(shape, lambda i: (0, 0))
    row_block = pl.BlockSpec((TB, D), lambda i: (i, 0))
    idx_block = pl.BlockSpec((TB // 128, 128), lambda i: (i, 0))

    out, it, ie, i0, i1, losssum = pl.pallas_call(
        _body,
        grid=(GRID,),
        in_specs=[
            row_block,
            whole((D, D)), whole((1, D)),
            whole((D, D)), whole((1, D)),
            whole((D, D)), whole((1, D)),
            whole((3 * D, D)), whole((1, D)),
            whole((KT, D)), whole((KE, D)),
            whole((KD, D)), whole((KD, D)),
        ],
        out_specs=[
            row_block,
            idx_block, idx_block, idx_block, idx_block,
            pl.BlockSpec((1, 1), lambda i: (0, 0)),
        ],
        out_shape=[
            jax.ShapeDtypeStruct((N, D), _F32),
            jax.ShapeDtypeStruct((N // 128, 128), jnp.int32),
            jax.ShapeDtypeStruct((N // 128, 128), jnp.int32),
            jax.ShapeDtypeStruct((N // 128, 128), jnp.int32),
            jax.ShapeDtypeStruct((N // 128, 128), jnp.int32),
            jax.ShapeDtypeStruct((1, 1), _F32),
        ],
        compiler_params=pltpu.CompilerParams(
            dimension_semantics=("arbitrary",)),
    )(xf, tw, tb.reshape(1, D), ew, eb.reshape(1, D),
      dw, db.reshape(1, D), ow, ob.reshape(1, D),
      cb_t, cb_e, cb_d0, cb_d1)

    loss = (1.0 + COMMIT) * losssum[0, 0] / jnp.float32(N * D)
    return (out.reshape(B, T, D),
            it.reshape(B, T), ie.reshape(B, T),
            i0.reshape(B, T), i1.reshape(B, T),
            loss)


# speedup vs baseline: 2.6303x; 1.8277x over previous
import jax
import jax.numpy as jnp
from jax.experimental import pallas as pl
from jax.experimental.pallas import tpu as pltpu

B, T, D = 8, 2048, 512
N = B * T
KT, KE, KD = 1024, 512, 1024
TB = 1024
GRID = N // TB
COMMIT = 0.25

_F32 = jnp.float32


def _vq_argmin(f, cb):
    rowsq = jnp.sum(f * f, axis=1, keepdims=True)
    cbsq = jnp.sum(cb * cb, axis=1)
    s = jax.lax.dot_general(f, cb, (((1,), (1,)), ((), ())),
                            preferred_element_type=_F32)
    d = rowsq + cbsq[None, :] - 2.0 * s
    dmin = jnp.min(d, axis=1, keepdims=True)
    iota = jax.lax.broadcasted_iota(jnp.int32, d.shape, 1)
    idx = jnp.min(jnp.where(d == dmin, iota, jnp.int32(d.shape[1])), axis=1)
    oh = (iota == idx[:, None]).astype(_F32)
    return idx, oh, jnp.sum(dmin)


def _body(x_ref, tw_ref, tb_ref, ew_ref, eb_ref, dw_ref, db_ref,
          ow_ref, ob_ref, cbt_ref, cbe_ref, cbd0_ref, cbd1_ref,
          out_ref, it_ref, ie_ref, i0_ref, i1_ref, loss_ref,
          mt_ref, me_ref, md1_ref):
    @pl.when(pl.program_id(0) == 0)
    def _precompute():
        ow = ow_ref[...]
        mt_ref[...] = jnp.dot(cbt_ref[...], ow[0:D],
                              preferred_element_type=_F32)
        me_ref[...] = jnp.dot(cbe_ref[...], ow[D:2 * D],
                              preferred_element_type=_F32)
        md1_ref[...] = jnp.dot(cbd1_ref[...], ow[2 * D:3 * D],
                               preferred_element_type=_F32)

    xb = x_ref[...]
    tf = jnp.dot(xb, tw_ref[...], preferred_element_type=_F32) + tb_ref[...]
    ef = jnp.dot(xb, ew_ref[...], preferred_element_type=_F32) + eb_ref[...]
    df = jnp.dot(xb, dw_ref[...], preferred_element_type=_F32) + db_ref[...]

    idx_t, oh_t, l_t = _vq_argmin(tf, cbt_ref[...])
    idx_e, oh_e, l_e = _vq_argmin(ef, cbe_ref[...])

    idx0, oh0, l_0 = _vq_argmin(df, cbd0_ref[...])
    q0 = jax.lax.dot_general(oh0, cbd0_ref[...], (((1,), (0,)), ((), ())),
                             preferred_element_type=_F32,
                             precision=jax.lax.Precision.HIGHEST)
    q0st = df + (q0 - df)
    r = df - q0st
    idx1, oh1, l_1 = _vq_argmin(r, cbd1_ref[...])

    out = (jax.lax.dot_general(oh_t, mt_ref[...], (((1,), (0,)), ((), ())),
                               preferred_element_type=_F32)
           + jax.lax.dot_general(oh_e, me_ref[...], (((1,), (0,)), ((), ())),
                                 preferred_element_type=_F32)
           + jax.lax.dot_general(oh1, md1_ref[...], (((1,), (0,)), ((), ())),
                                 preferred_element_type=_F32)
           + jnp.dot(q0st, ow_ref[2 * D:3 * D], preferred_element_type=_F32)
           + ob_ref[...])
    out_ref[...] = out

    it_ref[...] = idx_t.reshape(TB // 128, 128)
    ie_ref[...] = idx_e.reshape(TB // 128, 128)
    i0_ref[...] = idx0.reshape(TB // 128, 128)
    i1_ref[...] = idx1.reshape(TB // 128, 128)

    part2d = (l_t + l_e + l_0 + l_1).reshape(1, 1)

    @pl.when(pl.program_id(0) == 0)
    def _init():
        loss_ref[...] = part2d

    @pl.when(pl.program_id(0) != 0)
    def _acc():
        loss_ref[...] += part2d


def kernel(x, tw, tb, ew, eb, dw, db, ow, ob, cb_t, cb_e, cb_d0, cb_d1):
    xf = x.reshape(N, D)
    whole = lambda shape: pl.BlockSpec(shape, lambda i: (0, 0))
    row_block = pl.BlockSpec((TB, D), lambda i: (i, 0))
    idx_block = pl.BlockSpec((TB // 128, 128), lambda i: (i, 0))

    out, it, ie, i0, i1, losssum = pl.pallas_call(
        _body,
        grid=(GRID,),
        in_specs=[
            row_block,
            whole((D, D)), whole((1, D)),
            whole((D, D)), whole((1, D)),
            whole((D, D)), whole((1, D)),
            whole((3 * D, D)), whole((1, D)),
            whole((KT, D)), whole((KE, D)),
            whole((KD, D)), whole((KD, D)),
        ],
        out_specs=[
            row_block,
            idx_block, idx_block, idx_block, idx_block,
            pl.BlockSpec((1, 1), lambda i: (0, 0)),
        ],
        out_shape=[
            jax.ShapeDtypeStruct((N, D), _F32),
            jax.ShapeDtypeStruct((N // 128, 128), jnp.int32),
            jax.ShapeDtypeStruct((N // 128, 128), jnp.int32),
            jax.ShapeDtypeStruct((N // 128, 128), jnp.int32),
            jax.ShapeDtypeStruct((N // 128, 128), jnp.int32),
            jax.ShapeDtypeStruct((1, 1), _F32),
        ],
        scratch_shapes=[
            pltpu.VMEM((KT, D), _F32),
            pltpu.VMEM((KE, D), _F32),
            pltpu.VMEM((KD, D), _F32),
        ],
        compiler_params=pltpu.CompilerParams(
            dimension_semantics=("arbitrary",)),
    )(xf, tw, tb.reshape(1, D), ew, eb.reshape(1, D),
      dw, db.reshape(1, D), ow, ob.reshape(1, D),
      cb_t, cb_e, cb_d0, cb_d1)

    loss = (1.0 + COMMIT) * losssum[0, 0] / jnp.float32(N * D)
    return (out.reshape(B, T, D),
            it.reshape(B, T), ie.reshape(B, T),
            i0.reshape(B, T), i1.reshape(B, T),
            loss)
